# baseline (speedup 1.0000x reference)
import jax
import jax.numpy as jnp
from jax import lax
from jax.experimental import pallas as pl
from jax.experimental.pallas import tpu as pltpu, tpu_sc as plsc

B = 16
MAX_LEN = 2048
D = 512
TOTAL = 16384
NW = 32
ROWS_PER_W = (B * MAX_LEN) // NW
C = 64
NCHUNK = ROWS_PER_W // C
LANES = 16


def _unrag_body(cu_hbm, flat_hbm, zeros_hbm, out_hbm,
                cu_v, buf, zbuf, gsem, wsem, zsem):
    cid = lax.axis_index("c")
    sid = lax.axis_index("s")
    wid = sid * 2 + cid
    b = wid // 2
    j0 = (wid % 2) * (MAX_LEN // 2)
    obase = wid * ROWS_PER_W

    pltpu.sync_copy(cu_hbm, cu_v)
    seg_start = cu_v[pl.ds(b, LANES)][0]
    seg_end = cu_v[pl.ds(b + 1, LANES)][0]

    start = seg_start + j0
    n = jnp.clip(seg_end - start, 0, ROWS_PER_W)
    nfull = n // C
    rem = n - nfull * C
    nceil = nfull + (rem > 0).astype(jnp.int32)

    pltpu.sync_copy(zeros_hbm, zbuf)

    def zero_chunk(c, carry):
        pltpu.make_async_copy(zbuf, out_hbm.at[pl.ds(obase + c * C, C)], zsem).start()
        return carry

    lax.fori_loop(nceil, NCHUNK, zero_chunk, 0)

    lane = lax.iota(jnp.int32, LANES)
    hi = seg_end - 1

    def gather_chunk(c):
        ph = lax.rem(c, 2)
        for q in range(C // LANES):
            idx = jnp.minimum(start + c * C + q * LANES + lane, hi)
            pltpu.make_async_copy(flat_hbm.at[idx],
                                  buf.at[ph, pl.ds(q * LANES, LANES)],
                                  gsem).start()

    @pl.when(nceil > 0)
    def _prime():
        gather_chunk(0)

    def data_chunk(c, carry):
        ph = lax.rem(c, 2)
        for q in range(C // LANES):
            idx = jnp.minimum(start + c * C + q * LANES + lane, hi)
            pltpu.make_async_copy(flat_hbm.at[idx],
                                  buf.at[ph, pl.ds(q * LANES, LANES)],
                                  gsem).wait()

        @pl.when(c >= 1)
        def _lag():
            pltpu.make_async_copy(buf.at[lax.rem(c + 1, 2)],
                                  out_hbm.at[pl.ds(obase + (c - 1) * C, C)],
                                  wsem).wait()

        @pl.when(c + 1 < nceil)
        def _next():
            gather_chunk(c + 1)

        @pl.when((c + 1 == nceil) & (rem > 0))
        def _zero_tail():
            zrow = jnp.zeros((LANES,), jnp.float32)

            def ztail(i, carry):
                for k in range(D // LANES):
                    buf[ph, i, pl.ds(k * LANES, LANES)] = zrow
                return carry

            lax.fori_loop(rem, C, ztail, 0)

        pltpu.make_async_copy(buf.at[ph], out_hbm.at[pl.ds(obase + c * C, C)],
                              wsem).start()
        return carry

    lax.fori_loop(0, nceil, data_chunk, 0)

    @pl.when(nceil > 0)
    def _write_drain():
        c = nceil - 1
        pltpu.make_async_copy(buf.at[lax.rem(c, 2)],
                              out_hbm.at[pl.ds(obase + c * C, C)], wsem).wait()

    def zero_drain(c, carry):
        pltpu.make_async_copy(zbuf, out_hbm.at[pl.ds(obase + c * C, C)], zsem).wait()
        return carry

    lax.fori_loop(nceil, NCHUNK, zero_drain, 0)


def kernel(flat, cu_seqlens):
    cu = cu_seqlens.astype(jnp.int32)
    cu = cu.at[0].set(0).at[-1].set(TOTAL)
    cu_pad = jnp.zeros((2 * LANES,), jnp.int32).at[:B + 1].set(cu)
    zeros_blk = jnp.zeros((C, D), jnp.float32)

    run = pl.kernel(
        _unrag_body,
        out_type=jax.ShapeDtypeStruct((B * MAX_LEN, D), jnp.float32),
        mesh=plsc.VectorSubcoreMesh(core_axis_name="c", subcore_axis_name="s"),
        scratch_types=[
            pltpu.VMEM((2 * LANES,), jnp.int32),
            pltpu.VMEM((2, C, D), jnp.float32),
            pltpu.VMEM((C, D), jnp.float32),
            pltpu.SemaphoreType.DMA,
            pltpu.SemaphoreType.DMA,
            pltpu.SemaphoreType.DMA,
        ],
    )
    dense = run(cu_pad, flat, zeros_blk)
    return dense.reshape(B, MAX_LEN, D)


# speedup vs baseline: 1.0425x; 1.0425x over previous
import jax
import jax.numpy as jnp
from jax import lax
from jax.experimental import pallas as pl
from jax.experimental.pallas import tpu as pltpu, tpu_sc as plsc

B = 16
MAX_LEN = 2048
D = 512
TOTAL = 16384
NW = 32
C = 64
NCB = MAX_LEN // C
LANES = 16


def _unrag_body(cu_hbm, flat_hbm, out_hbm, cu_v, idx0, idx1, buf, zbuf,
                gsem0, gsem1, wsem0, wsem1, zsem):
    cid = lax.axis_index("c")
    sid = lax.axis_index("s")
    w = sid * 2 + cid

    pltpu.sync_copy(cu_hbm, cu_v)
    lane = lax.iota(jnp.int32, LANES)
    zrow = jnp.zeros((LANES,), jnp.float32)

    def zinit(i, carry):
        for k in range(D // LANES):
            zbuf[i, pl.ds(k * LANES, LANES)] = zrow
        return carry

    lax.fori_loop(0, C, zinit, 0)

    def chunk_params(t):
        seg_start = cu_v[pl.ds(t, LANES)][0]
        seg_end = cu_v[pl.ds(t + 1, LANES)][0]
        cb = lax.rem(w + t, NCB)
        st = seg_start + cb * C
        cnt = jnp.clip(seg_end - st, 0, C)
        dst_off = t * MAX_LEN + cb * C
        return st, cnt, seg_end, dst_off

    def fire_gather(t):
        st, cnt, seg_end, _ = chunk_params(t)

        @pl.when(cnt > 0)
        def _():
            hi = seg_end - 1
            s = lax.rem(t, 2)

            @pl.when(s == 0)
            def _():
                for q in range(C // LANES):
                    idx0[pl.ds(q * LANES, LANES)] = jnp.minimum(
                        st + q * LANES + lane, hi)
                pltpu.make_async_copy(flat_hbm.at[idx0], buf.at[0], gsem0).start()

            @pl.when(s == 1)
            def _():
                for q in range(C // LANES):
                    idx1[pl.ds(q * LANES, LANES)] = jnp.minimum(
                        st + q * LANES + lane, hi)
                pltpu.make_async_copy(flat_hbm.at[idx1], buf.at[1], gsem1).start()

    def wait_write(slot, lt):
        @pl.when(lt >= 0)
        def _():
            _, _, _, dst_off = chunk_params(lt)
            dst = out_hbm.at[pl.ds(dst_off, C)]
            @pl.when(slot == 0)
            def _():
                pltpu.make_async_copy(buf.at[0], dst, wsem0).wait()
            @pl.when(slot == 1)
            def _():
                pltpu.make_async_copy(buf.at[1], dst, wsem1).wait()

    fire_gather(0)

    def body(t, carry):
        lt0, lt1 = carry
        st, cnt, seg_end, dst_off = chunk_params(t)
        s = lax.rem(t, 2)
        dst = out_hbm.at[pl.ds(dst_off, C)]

        @pl.when(cnt == 0)
        def _zero():
            pltpu.make_async_copy(zbuf, dst, zsem).start()

        @pl.when(cnt > 0)
        def _wait_gather():
            hi = seg_end - 1
            @pl.when(s == 0)
            def _():
                pltpu.make_async_copy(flat_hbm.at[idx0], buf.at[0], gsem0).wait()
            @pl.when(s == 1)
            def _():
                pltpu.make_async_copy(flat_hbm.at[idx1], buf.at[1], gsem1).wait()

        @pl.when(t + 1 < B)
        def _next():
            _, cnt1, _, _ = chunk_params(t + 1)

            @pl.when(cnt1 > 0)
            def _():
                s1 = lax.rem(t + 1, 2)
                wait_write(s1, jnp.where(s1 == 0, lt0, lt1))
                fire_gather(t + 1)

        @pl.when(cnt > 0)
        def _emit():
            @pl.when(cnt < C)
            def _tail():
                def ztail(i, carry2):
                    for k in range(D // LANES):
                        @pl.when(s == 0)
                        def _():
                            buf[0, i, pl.ds(k * LANES, LANES)] = zrow
                        @pl.when(s == 1)
                        def _():
                            buf[1, i, pl.ds(k * LANES, LANES)] = zrow
                    return carry2

                lax.fori_loop(cnt, C, ztail, 0)

            @pl.when(s == 0)
            def _():
                pltpu.make_async_copy(buf.at[0], dst, wsem0).start()
            @pl.when(s == 1)
            def _():
                pltpu.make_async_copy(buf.at[1], dst, wsem1).start()

        nlt0 = jnp.where((cnt > 0) & (s == 0), t, lt0)
        nlt1 = jnp.where((cnt > 0) & (s == 1), t, lt1)
        return nlt0, nlt1

    lt0, lt1 = lax.fori_loop(0, B, body, (jnp.int32(-1), jnp.int32(-1)))

    wait_write(jnp.int32(0), lt0)
    wait_write(jnp.int32(1), lt1)

    def zdrain(t, carry):
        _, cnt, _, dst_off = chunk_params(t)

        @pl.when(cnt == 0)
        def _():
            pltpu.make_async_copy(zbuf, out_hbm.at[pl.ds(dst_off, C)], zsem).wait()
        return carry

    lax.fori_loop(0, B, zdrain, 0)


def kernel(flat, cu_seqlens):
    cu = cu_seqlens.astype(jnp.int32)
    cu = cu.at[0].set(0).at[-1].set(TOTAL)
    cu_pad = jnp.zeros((2 * LANES,), jnp.int32).at[:B + 1].set(cu)

    run = pl.kernel(
        _unrag_body,
        out_type=jax.ShapeDtypeStruct((B * MAX_LEN, D), jnp.float32),
        mesh=plsc.VectorSubcoreMesh(core_axis_name="c", subcore_axis_name="s"),
        scratch_types=[
            pltpu.VMEM((2 * LANES,), jnp.int32),
            pltpu.VMEM((C,), jnp.int32),
            pltpu.VMEM((C,), jnp.int32),
            pltpu.VMEM((2, C, D), jnp.float32),
            pltpu.VMEM((C, D), jnp.float32),
            pltpu.SemaphoreType.DMA,
            pltpu.SemaphoreType.DMA,
            pltpu.SemaphoreType.DMA,
            pltpu.SemaphoreType.DMA,
            pltpu.SemaphoreType.DMA,
        ],
    )
    dense = run(cu_pad, flat)
    return dense.reshape(B, MAX_LEN, D)


# speedup vs baseline: 1.0467x; 1.0041x over previous
import jax
import jax.numpy as jnp
from jax import lax
from jax.experimental import pallas as pl
from jax.experimental.pallas import tpu as pltpu, tpu_sc as plsc

B = 16
MAX_LEN = 2048
D = 512
TOTAL = 16384
NW = 32
C = 64
NCB = MAX_LEN // C
LANES = 16


def _unrag_body(cu_hbm, flat_hbm, out_hbm, cu_v, idx0, idx1, buf, zbuf,
                gsem0, gsem1, wsem0, wsem1, zsem):
    cid = lax.axis_index("c")
    sid = lax.axis_index("s")
    w = sid * 2 + cid

    pltpu.sync_copy(cu_hbm, cu_v.at[pl.ds(0, B + 1)])
    lane = lax.iota(jnp.int32, LANES)
    zrow = jnp.zeros((LANES,), jnp.float32)

    def zinit(i, carry):
        for k in range(D // LANES):
            zbuf[i, pl.ds(k * LANES, LANES)] = zrow
        return carry

    lax.fori_loop(0, C, zinit, 0)

    def chunk_params(t):
        seg_start = jnp.where(t == 0, 0, cu_v[pl.ds(t, LANES)][0])
        seg_end = jnp.where(t == B - 1, TOTAL, cu_v[pl.ds(t + 1, LANES)][0])
        cb = lax.rem(w + t, NCB)
        st = seg_start + cb * C
        cnt = jnp.clip(seg_end - st, 0, C)
        dst_off = t * MAX_LEN + cb * C
        return st, cnt, seg_end, dst_off

    def fire_gather(t):
        st, cnt, seg_end, _ = chunk_params(t)

        @pl.when(cnt > 0)
        def _():
            hi = seg_end - 1
            s = lax.rem(t, 2)

            @pl.when(s == 0)
            def _():
                for q in range(C // LANES):
                    idx0[pl.ds(q * LANES, LANES)] = jnp.minimum(
                        st + q * LANES + lane, hi)
                pltpu.make_async_copy(flat_hbm.at[idx0], buf.at[0], gsem0).start()

            @pl.when(s == 1)
            def _():
                for q in range(C // LANES):
                    idx1[pl.ds(q * LANES, LANES)] = jnp.minimum(
                        st + q * LANES + lane, hi)
                pltpu.make_async_copy(flat_hbm.at[idx1], buf.at[1], gsem1).start()

    def wait_write(slot, lt):
        @pl.when(lt >= 0)
        def _():
            _, _, _, dst_off = chunk_params(lt)
            dst = out_hbm.at[pl.ds(dst_off, C)]
            @pl.when(slot == 0)
            def _():
                pltpu.make_async_copy(buf.at[0], dst, wsem0).wait()
            @pl.when(slot == 1)
            def _():
                pltpu.make_async_copy(buf.at[1], dst, wsem1).wait()

    fire_gather(0)

    def body(t, carry):
        lt0, lt1 = carry
        st, cnt, seg_end, dst_off = chunk_params(t)
        s = lax.rem(t, 2)
        dst = out_hbm.at[pl.ds(dst_off, C)]

        @pl.when(cnt == 0)
        def _zero():
            pltpu.make_async_copy(zbuf, dst, zsem).start()

        @pl.when(cnt > 0)
        def _wait_gather():
            hi = seg_end - 1
            @pl.when(s == 0)
            def _():
                pltpu.make_async_copy(flat_hbm.at[idx0], buf.at[0], gsem0).wait()
            @pl.when(s == 1)
            def _():
                pltpu.make_async_copy(flat_hbm.at[idx1], buf.at[1], gsem1).wait()

        @pl.when(t + 1 < B)
        def _next():
            _, cnt1, _, _ = chunk_params(t + 1)

            @pl.when(cnt1 > 0)
            def _():
                s1 = lax.rem(t + 1, 2)
                wait_write(s1, jnp.where(s1 == 0, lt0, lt1))
                fire_gather(t + 1)

        @pl.when(cnt > 0)
        def _emit():
            @pl.when(cnt < C)
            def _tail():
                def ztail(i, carry2):
                    for k in range(D // LANES):
                        @pl.when(s == 0)
                        def _():
                            buf[0, i, pl.ds(k * LANES, LANES)] = zrow
                        @pl.when(s == 1)
                        def _():
                            buf[1, i, pl.ds(k * LANES, LANES)] = zrow
                    return carry2

                lax.fori_loop(cnt, C, ztail, 0)

            @pl.when(s == 0)
            def _():
                pltpu.make_async_copy(buf.at[0], dst, wsem0).start()
            @pl.when(s == 1)
            def _():
                pltpu.make_async_copy(buf.at[1], dst, wsem1).start()

        nlt0 = jnp.where((cnt > 0) & (s == 0), t, lt0)
        nlt1 = jnp.where((cnt > 0) & (s == 1), t, lt1)
        return nlt0, nlt1

    lt0, lt1 = lax.fori_loop(0, B, body, (jnp.int32(-1), jnp.int32(-1)))

    wait_write(jnp.int32(0), lt0)
    wait_write(jnp.int32(1), lt1)

    def zdrain(t, carry):
        _, cnt, _, dst_off = chunk_params(t)

        @pl.when(cnt == 0)
        def _():
            pltpu.make_async_copy(zbuf, out_hbm.at[pl.ds(dst_off, C)], zsem).wait()
        return carry

    lax.fori_loop(0, B, zdrain, 0)


def kernel(flat, cu_seqlens):
    cu = cu_seqlens.astype(jnp.int32)

    run = pl.kernel(
        _unrag_body,
        out_type=jax.ShapeDtypeStruct((B * MAX_LEN, D), jnp.float32),
        mesh=plsc.VectorSubcoreMesh(core_axis_name="c", subcore_axis_name="s"),
        scratch_types=[
            pltpu.VMEM((2 * LANES,), jnp.int32),
            pltpu.VMEM((C,), jnp.int32),
            pltpu.VMEM((C,), jnp.int32),
            pltpu.VMEM((2, C, D), jnp.float32),
            pltpu.VMEM((C, D), jnp.float32),
            pltpu.SemaphoreType.DMA,
            pltpu.SemaphoreType.DMA,
            pltpu.SemaphoreType.DMA,
            pltpu.SemaphoreType.DMA,
            pltpu.SemaphoreType.DMA,
        ],
    )
    dense = run(cu, flat)
    return dense.reshape(B, MAX_LEN, D)
